# Initial kernel scaffold; baseline (speedup 1.0000x reference)
import jax
import jax.numpy as jnp
from jax.experimental import pallas as pl


def kernel(x, edge_index, edge_weight, W0, W1, b1, W2, b2):
    raise NotImplementedError("write your pallas kernel here")


# speedup vs baseline: 3.8055x; 3.8055x over previous
import functools

import jax
import jax.numpy as jnp
from jax import lax
from jax.experimental import pallas as pl
from jax.experimental.pallas import tpu as pltpu
from jax.experimental.pallas import tpu_sc as plsc

N = 10000
DF = 128
DH = 512
NLAB = 10

NC = 2
NS = 16
NW = NC * NS
CHUNK = 128
CPW = 80
EPW = CHUNK * CPW
EPAD = NW * EPW
NPAD = 10240
RPT = NPAD // NS


def _prep_edges(edge_index, edge_weight):
    src = edge_index[0].astype(jnp.int32)
    dst = edge_index[1].astype(jnp.int32)
    w = edge_weight.astype(jnp.float32)
    pad = EPAD - src.shape[0]
    packed = src | (dst << 16)
    packed = jnp.pad(packed, (0, pad))
    w = jnp.pad(w, (0, pad))
    return (packed.reshape(NW, CPW, 1, CHUNK),
            w.reshape(NW, CPW, 1, CHUNK))


def _sc_segment_sum(x, ed, wg):
    mesh = plsc.VectorSubcoreMesh(core_axis_name="c", subcore_axis_name="s")

    @functools.partial(
        pl.kernel,
        out_type=jax.ShapeDtypeStruct((NC, NPAD, DF), jnp.float32),
        mesh=mesh,
        scratch_types=[
            pltpu.VMEM((1, CHUNK), jnp.int32),
            pltpu.VMEM((1, CHUNK), jnp.int32),
            pltpu.VMEM((1, CHUNK), jnp.float32),
            pltpu.VMEM((1, CHUNK), jnp.float32),
            pltpu.VMEM((1, CHUNK), jnp.int32),
            pltpu.VMEM((1, CHUNK), jnp.int32),
            pltpu.VMEM((1, CHUNK), jnp.int32),
            pltpu.VMEM((1, CHUNK), jnp.int32),
            pltpu.VMEM((CHUNK, DF), jnp.float32),
            pltpu.VMEM((CHUNK, DF), jnp.float32),
            pltpu.VMEM_SHARED((NPAD, DF), jnp.float32),
            pltpu.SemaphoreType.DMA,
            pltpu.SemaphoreType.DMA,
            pltpu.SemaphoreType.DMA,
            pltpu.SemaphoreType.DMA,
            pltpu.SemaphoreType.DMA,
            pltpu.SemaphoreType.DMA,
        ],
    )
    def k(x_hbm, ed_hbm, wg_hbm, out_hbm,
          edA, edB, wgA, wgB, srcA, srcB, dstA, dstB, rowsA, rowsB, acc,
          gsemA, gsemB, esemA, esemB, wsemA, wsemB):
        cid = lax.axis_index("c")
        sid = lax.axis_index("s")
        wid = cid * NS + sid

        def zrow(r, carry):
            for j in range(DF // 16):
                rowsA[r, pl.ds(16 * j, 16)] = jnp.zeros((16,), jnp.float32)
            return carry
        lax.fori_loop(0, CHUNK, zrow, 0)
        row0 = sid * RPT
        for kk in range(RPT // CHUNK):
            pltpu.sync_copy(rowsA, acc.at[pl.ds(row0 + kk * CHUNK, CHUNK)])
        plsc.subcore_barrier()

        def e_issue(c, edb, esem, wgb, wsem):
            pltpu.async_copy(ed_hbm.at[wid, c], edb, esem)
            pltpu.async_copy(wg_hbm.at[wid, c], wgb, wsem)

        def e_wait(c, edb, esem, wgb, wsem):
            pltpu.make_async_copy(ed_hbm.at[wid, c], edb, esem).wait()
            pltpu.make_async_copy(wg_hbm.at[wid, c], wgb, wsem).wait()

        def unpack(edb, srcb, dstb):
            for g in range(CHUNK // 16):
                v = edb[0, pl.ds(16 * g, 16)]
                srcb[0, pl.ds(16 * g, 16)] = v & 0xFFFF
                dstb[0, pl.ds(16 * g, 16)] = v >> 16

        def g_issue(srcb, rows, gsem):
            pltpu.async_copy(x_hbm.at[srcb.at[0]], rows, gsem)

        def g_wait(srcb, rows, gsem):
            pltpu.make_async_copy(x_hbm.at[srcb.at[0]], rows, gsem).wait()

        def scale_and_scatter(wgb, dstb, rows):
            def gbody(g, carry):
                wv = wgb[0, pl.ds(16 * g, 16)]
                for l in range(16):
                    e = g * 16 + l
                    s = wv[l]
                    for j in range(DF // 16):
                        rows[e, pl.ds(16 * j, 16)] = (
                            rows[e, pl.ds(16 * j, 16)] * s)
                return carry
            lax.fori_loop(0, CHUNK // 16, gbody, 0)
            pltpu.sync_copy(rows, acc.at[dstb.at[0]], add=True)

        e_issue(0, edA, esemA, wgA, wsemA)
        e_wait(0, edA, esemA, wgA, wsemA)
        unpack(edA, srcA, dstA)
        g_issue(srcA, rowsA, gsemA)
        e_issue(1, edB, esemB, wgB, wsemB)

        bufs = ((edA, wgA, srcA, dstA, rowsA, gsemA, esemA, wsemA),
                (edB, wgB, srcB, dstB, rowsB, gsemB, esemB, wsemB))

        def pair(cc, carry):
            for par in range(2):
                c = 2 * cc + par
                edP, wgP, srcP, dstP, rowsP, gsemP, esemP, wsemP = bufs[par]
                edQ, wgQ, srcQ, dstQ, rowsQ, gsemQ, esemQ, wsemQ = bufs[1 - par]
                g_wait(srcP, rowsP, gsemP)

                @pl.when(c + 1 < CPW)
                def _():
                    e_wait(c + 1, edQ, esemQ, wgQ, wsemQ)
                    unpack(edQ, srcQ, dstQ)
                    g_issue(srcQ, rowsQ, gsemQ)

                scale_and_scatter(wgP, dstP, rowsP)

                @pl.when(c + 2 < CPW)
                def _():
                    e_issue(c + 2, edP, esemP, wgP, wsemP)
            return carry
        lax.fori_loop(0, CPW // 2, pair, 0)

        plsc.subcore_barrier()

        for kk in range(RPT // CHUNK):
            pltpu.sync_copy(acc.at[pl.ds(row0 + kk * CHUNK, CHUNK)],
                            out_hbm.at[cid, pl.ds(row0 + kk * CHUNK, CHUNK)])

    return k(x, ed, wg)


def _dense_head(x, t1p, W0, W1, b1, W2, b2):
    blk = 1000
    grid = N // blk

    def body(x_ref, t_ref, w0_ref, w1_ref, b1_ref, w2_ref, b2_ref,
             out_ref, acc_ref):
        i = pl.program_id(0)
        t1 = t_ref[0] + t_ref[1]
        h = jnp.dot(x_ref[...], w0_ref[...],
                    preferred_element_type=jnp.float32)
        h = h + jnp.dot(t1, w1_ref[...],
                        preferred_element_type=jnp.float32)
        h = jnp.maximum(h + b1_ref[...], 0.0)
        ps = jnp.sum(h, axis=0, keepdims=True)

        @pl.when(i == 0)
        def _():
            acc_ref[...] = ps

        @pl.when(i > 0)
        def _():
            acc_ref[...] = acc_ref[...] + ps

        @pl.when(i == grid - 1)
        def _():
            logits = jnp.dot(acc_ref[...], w2_ref[...],
                             preferred_element_type=jnp.float32) + b2_ref[...]
            m = jnp.max(logits, axis=-1, keepdims=True)
            e = jnp.exp(logits - m)
            out_ref[...] = e / jnp.sum(e, axis=-1, keepdims=True)

    return pl.pallas_call(
        body,
        grid=(grid,),
        in_specs=[
            pl.BlockSpec((blk, DF), lambda i: (i, 0)),
            pl.BlockSpec((NC, blk, DF), lambda i: (0, i, 0)),
            pl.BlockSpec((DF, DH), lambda i: (0, 0)),
            pl.BlockSpec((DF, DH), lambda i: (0, 0)),
            pl.BlockSpec((1, DH), lambda i: (0, 0)),
            pl.BlockSpec((DH, NLAB), lambda i: (0, 0)),
            pl.BlockSpec((1, NLAB), lambda i: (0, 0)),
        ],
        out_specs=pl.BlockSpec((1, NLAB), lambda i: (0, 0)),
        out_shape=jax.ShapeDtypeStruct((1, NLAB), jnp.float32),
        scratch_shapes=[pltpu.VMEM((1, DH), jnp.float32)],
    )(x, t1p, W0, W1, b1.reshape(1, DH), W2, b2.reshape(1, NLAB))


def kernel(x, edge_index, edge_weight, W0, W1, b1, W2, b2):
    ed, wg = _prep_edges(edge_index, edge_weight)
    t1p = _sc_segment_sum(x, ed, wg)
    return _dense_head(x, t1p, W0, W1, b1, W2, b2)
